# initial kernel scaffold (unmeasured)
import jax
import jax.numpy as jnp
from jax import lax
from jax.experimental import pallas as pl
from jax.experimental.pallas import tpu as pltpu

B = 32
H = 16
D = 128
TOK = 32
NB = 256
NZ = 4
PAGES_LOCAL = 256
CHUNK_PAGES = 32
C = PAGES_LOCAL // CHUNK_PAGES
CHUNK_TOK = CHUNK_PAGES * TOK
SCALE = D ** -0.5
NEG_INF = -1e30


def kernel(Q, K, V, bt, lens):
    lens2d = lens.reshape(B, 1)

    def body(q_ref, k_ref, v_ref, bt_ref, lens_ref, out_ref,
             m_s, l_s, acc_s, packed_mine, packed_recv,
             send_sems, recv_sems):
        c = pl.program_id(0)
        my_x = lax.axis_index("x")
        my_y = lax.axis_index("y")
        my_z = lax.axis_index("z")

        @pl.when(c == 0)
        def _():
            m_s[:, :] = jnp.full((B, H), NEG_INF, jnp.float32)
            l_s[:, :] = jnp.zeros((B, H), jnp.float32)
            acc_s[:, :, :] = jnp.zeros((H, B, D), jnp.float32)

        pids = (my_z * PAGES_LOCAL + c * CHUNK_PAGES
                + lax.broadcasted_iota(jnp.int32, (1, CHUNK_PAGES, 1), 1))
        btx = bt_ref[:, :].reshape(B, 1, NB)
        j_iota = lax.broadcasted_iota(jnp.int32, (B, 1, NB), 2)
        valid = j_iota < lens_ref[:, :].reshape(B, 1, 1)
        hit = jnp.logical_and(btx == pids, valid).astype(jnp.float32)
        count_chunk = jnp.sum(hit, axis=-1)
        count_tok = jnp.broadcast_to(
            count_chunk[:, :, None], (B, CHUNK_PAGES, TOK)
        ).reshape(B, CHUNK_TOK)

        for h in range(H):
            qh = q_ref[:, 0, h, :].astype(jnp.bfloat16)
            kh = k_ref[:, :, h, :].reshape(CHUNK_TOK, D).astype(jnp.bfloat16)
            vh = v_ref[:, :, h, :].reshape(CHUNK_TOK, D).astype(jnp.bfloat16)
            s = lax.dot_general(
                qh, kh, (((1,), (1,)), ((), ())),
                preferred_element_type=jnp.float32,
            ) * SCALE
            m_prev = m_s[:, h:h + 1]
            m_new = jnp.maximum(m_prev, jnp.max(s, axis=1, keepdims=True))
            alpha = jnp.exp(m_prev - m_new)
            p = count_tok * jnp.exp(s - m_new)
            l_s[:, h:h + 1] = l_s[:, h:h + 1] * alpha + jnp.sum(
                p, axis=1, keepdims=True)
            pv = lax.dot_general(
                p.astype(jnp.bfloat16), vh, (((1,), (0,)), ((), ())),
                preferred_element_type=jnp.float32,
            )
            acc_s[h, :, :] = acc_s[h, :, :] * alpha + pv
            m_s[:, h:h + 1] = m_new

        @pl.when(c == C - 1)
        def _():
            for h in range(H):
                packed_mine[h, :, :] = acc_s[h, :, :]
            stats = jnp.zeros((B, D), jnp.float32)
            stats = stats.at[:, 0:H].set(m_s[:, :])
            stats = stats.at[:, H:2 * H].set(l_s[:, :])
            packed_mine[H, :, :] = stats

            rdmas = []
            for dz in (1, 2, 3):
                rdma = pltpu.make_async_remote_copy(
                    src_ref=packed_mine,
                    dst_ref=packed_recv.at[dz - 1],
                    send_sem=send_sems.at[dz - 1],
                    recv_sem=recv_sems.at[dz - 1],
                    device_id=(my_x, my_y, (my_z + dz) % NZ),
                    device_id_type=pl.DeviceIdType.MESH,
                )
                rdma.start()
                rdmas.append(rdma)
            for rdma in rdmas:
                rdma.wait_recv()

            m_parts = [m_s[:, :]] + [
                packed_recv[k, H, :, 0:H] for k in range(NZ - 1)]
            l_parts = [l_s[:, :]] + [
                packed_recv[k, H, :, H:2 * H] for k in range(NZ - 1)]
            m_tot = m_parts[0]
            for mp in m_parts[1:]:
                m_tot = jnp.maximum(m_tot, mp)
            scales = [jnp.exp(mp - m_tot) for mp in m_parts]
            l_tot = scales[0] * l_parts[0]
            for sc, lp in zip(scales[1:], l_parts[1:]):
                l_tot = l_tot + sc * lp
            for h in range(H):
                acc_tot = acc_s[h, :, :] * scales[0][:, h:h + 1]
                for k in range(NZ - 1):
                    acc_tot = acc_tot + (
                        packed_recv[k, h, :, :] * scales[k + 1][:, h:h + 1])
                out_ref[:, 0, h, :] = acc_tot / l_tot[:, h:h + 1]

            for rdma in rdmas:
                rdma.wait_send()

    grid = (C,)
    return pl.pallas_call(
        body,
        grid=grid,
        out_shape=jax.ShapeDtypeStruct((B, 1, H, D), jnp.float32),
        in_specs=[
            pl.BlockSpec((B, 1, H, D), lambda c: (0, 0, 0, 0)),
            pl.BlockSpec((CHUNK_PAGES, TOK, H, D), lambda c: (c, 0, 0, 0)),
            pl.BlockSpec((CHUNK_PAGES, TOK, H, D), lambda c: (c, 0, 0, 0)),
            pl.BlockSpec((B, NB), lambda c: (0, 0)),
            pl.BlockSpec((B, 1), lambda c: (0, 0)),
        ],
        out_specs=pl.BlockSpec((B, 1, H, D), lambda c: (0, 0, 0, 0)),
        scratch_shapes=[
            pltpu.VMEM((B, H), jnp.float32),
            pltpu.VMEM((B, H), jnp.float32),
            pltpu.VMEM((H, B, D), jnp.float32),
            pltpu.VMEM((H + 1, B, D), jnp.float32),
            pltpu.VMEM((NZ - 1, H + 1, B, D), jnp.float32),
            pltpu.SemaphoreType.DMA((NZ - 1,)),
            pltpu.SemaphoreType.DMA((NZ - 1,)),
        ],
        compiler_params=pltpu.CompilerParams(
            dimension_semantics=("arbitrary",),
            collective_id=0,
        ),
    )(Q, K, V, bt, lens2d)


# baseline (device time: 200624 ns/iter reference)
import jax
import jax.numpy as jnp
from jax import lax
from jax.experimental import pallas as pl
from jax.experimental.pallas import tpu as pltpu

B = 32
H = 16
D = 128
TOK = 32
NB = 256
NZ = 4
PAGES_LOCAL = 256
CHUNK_PAGES = 32
C = PAGES_LOCAL // CHUNK_PAGES
CHUNK_TOK = CHUNK_PAGES * TOK
SCALE = D ** -0.5
NEG_INF = -1e30


def kernel(Q, K, V, bt, lens):
    lens2d = lens.reshape(B, 1)

    def body(q_ref, k_ref, v_ref, bt_ref, lens_ref, out_ref,
             m_s, l_s, acc_s, packed_mine, packed_recv,
             send_sems, recv_sems):
        c = pl.program_id(0)
        my_x = lax.axis_index("x")
        my_y = lax.axis_index("y")
        my_z = lax.axis_index("z")

        @pl.when(c == 0)
        def _():
            m_s[:, :] = jnp.full((B, H), NEG_INF, jnp.float32)
            l_s[:, :] = jnp.zeros((B, H), jnp.float32)
            acc_s[:, :, :] = jnp.zeros((H, B, D), jnp.float32)

        pids = (my_z * PAGES_LOCAL + c * CHUNK_PAGES
                + lax.broadcasted_iota(jnp.int32, (1, CHUNK_PAGES, 1), 1))
        btx = bt_ref[:, :].reshape(B, 1, NB)
        j_iota = lax.broadcasted_iota(jnp.int32, (B, 1, NB), 2)
        valid = j_iota < lens_ref[:, :].reshape(B, 1, 1)
        hit = jnp.logical_and(btx == pids, valid).astype(jnp.float32)
        count_chunk = jnp.sum(hit, axis=-1)
        count_tok = jnp.broadcast_to(
            count_chunk[:, :, None], (B, CHUNK_PAGES, TOK)
        ).reshape(B, CHUNK_TOK)

        for h in range(H):
            qh = q_ref[:, 0, h, :].astype(jnp.bfloat16)
            kh = k_ref[:, :, h, :].reshape(CHUNK_TOK, D).astype(jnp.bfloat16)
            vh = v_ref[:, :, h, :].reshape(CHUNK_TOK, D).astype(jnp.bfloat16)
            s = lax.dot_general(
                qh, kh, (((1,), (1,)), ((), ())),
                preferred_element_type=jnp.float32,
            ) * SCALE
            m_prev = m_s[:, h:h + 1]
            m_new = jnp.maximum(m_prev, jnp.max(s, axis=1, keepdims=True))
            alpha = jnp.exp(m_prev - m_new)
            p = count_tok * jnp.exp(s - m_new)
            l_s[:, h:h + 1] = l_s[:, h:h + 1] * alpha + jnp.sum(
                p, axis=1, keepdims=True)
            pv = lax.dot_general(
                p.astype(jnp.bfloat16), vh, (((1,), (0,)), ((), ())),
                preferred_element_type=jnp.float32,
            )
            acc_s[h, :, :] = acc_s[h, :, :] * alpha + pv
            m_s[:, h:h + 1] = m_new

        @pl.when(c == C - 1)
        def _():
            for h in range(H):
                packed_mine[h, :, :] = acc_s[h, :, :]
            packed_mine[H, :, 0:H] = m_s[:, :]
            packed_mine[H, :, H:2 * H] = l_s[:, :]

            rdmas = []
            for dz in (1, 2, 3):
                rdma = pltpu.make_async_remote_copy(
                    src_ref=packed_mine,
                    dst_ref=packed_recv.at[dz - 1],
                    send_sem=send_sems.at[dz - 1],
                    recv_sem=recv_sems.at[dz - 1],
                    device_id=(my_x, my_y, (my_z + dz) % NZ),
                    device_id_type=pl.DeviceIdType.MESH,
                )
                rdma.start()
                rdmas.append(rdma)
            for rdma in rdmas:
                rdma.wait_recv()

            m_parts = [m_s[:, :]] + [
                packed_recv[k, H, :, 0:H] for k in range(NZ - 1)]
            l_parts = [l_s[:, :]] + [
                packed_recv[k, H, :, H:2 * H] for k in range(NZ - 1)]
            m_tot = m_parts[0]
            for mp in m_parts[1:]:
                m_tot = jnp.maximum(m_tot, mp)
            scales = [jnp.exp(mp - m_tot) for mp in m_parts]
            l_tot = scales[0] * l_parts[0]
            for sc, lp in zip(scales[1:], l_parts[1:]):
                l_tot = l_tot + sc * lp
            for h in range(H):
                acc_tot = acc_s[h, :, :] * scales[0][:, h:h + 1]
                for k in range(NZ - 1):
                    acc_tot = acc_tot + (
                        packed_recv[k, h, :, :] * scales[k + 1][:, h:h + 1])
                out_ref[:, 0, h, :] = acc_tot / l_tot[:, h:h + 1]

            for rdma in rdmas:
                rdma.wait_send()

    grid = (C,)
    return pl.pallas_call(
        body,
        grid=grid,
        out_shape=jax.ShapeDtypeStruct((B, 1, H, D), jnp.float32),
        in_specs=[
            pl.BlockSpec((B, 1, H, D), lambda c: (0, 0, 0, 0)),
            pl.BlockSpec((CHUNK_PAGES, TOK, H, D), lambda c: (c, 0, 0, 0)),
            pl.BlockSpec((CHUNK_PAGES, TOK, H, D), lambda c: (c, 0, 0, 0)),
            pl.BlockSpec((B, NB), lambda c: (0, 0)),
            pl.BlockSpec((B, 1), lambda c: (0, 0)),
        ],
        out_specs=pl.BlockSpec((B, 1, H, D), lambda c: (0, 0, 0, 0)),
        scratch_shapes=[
            pltpu.VMEM((B, H), jnp.float32),
            pltpu.VMEM((B, H), jnp.float32),
            pltpu.VMEM((H, B, D), jnp.float32),
            pltpu.VMEM((H + 1, B, D), jnp.float32),
            pltpu.VMEM((NZ - 1, H + 1, B, D), jnp.float32),
            pltpu.SemaphoreType.DMA((NZ - 1,)),
            pltpu.SemaphoreType.DMA((NZ - 1,)),
        ],
        compiler_params=pltpu.CompilerParams(
            dimension_semantics=("arbitrary",),
            vmem_limit_bytes=100 * 1024 * 1024,
        ),
    )(Q, K, V, bt, lens2d)
